# baseline (speedup 1.0000x reference)
import functools

import jax
import jax.numpy as jnp
from jax import lax
from jax.experimental import pallas as pl
from jax.experimental.pallas import tpu as pltpu
from jax.experimental.pallas import tpu_sc as plsc

N, E, D, ED, H, L = 10000, 320000, 128, 16, 128, 3
LANES = 16
NC, NS = 2, 16
NPAD = 10240
NW = NC * NS
EPT = E // NW
C = 80
ITERS = EPT // C


def _proj_body(x_ref, w_ref, b_ref, o_ref):
    o_ref[...] = (
        jax.lax.dot_general(x_ref[...], w_ref[...], (((1,), (0,)), ((), ())),
                            preferred_element_type=jnp.float32,
                            precision=jax.lax.Precision.HIGHEST)
        + b_ref[...]
    )


def _proj(x, w, b, blk):
    n = x.shape[0]
    return pl.pallas_call(
        _proj_body,
        grid=(n // blk,),
        in_specs=[
            pl.BlockSpec((blk, x.shape[1]), lambda i: (i, jnp.int32(0))),
            pl.BlockSpec(w.shape, lambda i: (jnp.int32(0), jnp.int32(0))),
            pl.BlockSpec((1, b.shape[1]), lambda i: (jnp.int32(0), jnp.int32(0))),
        ],
        out_specs=pl.BlockSpec((blk, w.shape[1]), lambda i: (i, jnp.int32(0))),
        out_shape=jax.ShapeDtypeStruct((n, w.shape[1]), jnp.float32),
    )(x, w, b)


def _node_body(h_ref, a0_ref, a1_ref, scale_ref, w1_ref, b1_ref, w2_ref,
               b2_ref, g_ref, be_ref, o_ref):
    z = scale_ref[0, 0] * h_ref[...] + a0_ref[...] + a1_ref[...]
    t = jax.lax.dot_general(z, w1_ref[...], (((1,), (0,)), ((), ())),
                            preferred_element_type=jnp.float32,
                            precision=jax.lax.Precision.HIGHEST)
    t = jnp.maximum(t + b1_ref[...], 0.0)
    y = jax.lax.dot_general(t, w2_ref[...], (((1,), (0,)), ((), ())),
                            preferred_element_type=jnp.float32,
                            precision=jax.lax.Precision.HIGHEST)
    y = y + b2_ref[...]
    mu = jnp.mean(y, axis=-1, keepdims=True)
    var = jnp.mean((y - mu) ** 2, axis=-1, keepdims=True)
    y = (y - mu) * jax.lax.rsqrt(var + 1e-5) * g_ref[...] + be_ref[...]
    o_ref[...] = jnp.maximum(y, 0.0)


def _node_update(h, a0, a1, scale, w1, b1, w2, b2, gamma, beta, blk):
    n = h.shape[0]
    return pl.pallas_call(
        _node_body,
        grid=(n // blk,),
        in_specs=[
            pl.BlockSpec((blk, H), lambda i: (i, jnp.int32(0))),
            pl.BlockSpec((blk, H), lambda i: (i, jnp.int32(0))),
            pl.BlockSpec((blk, H), lambda i: (i, jnp.int32(0))),
            pl.BlockSpec((1, 1), lambda i: (jnp.int32(0), jnp.int32(0)),
                         memory_space=pltpu.SMEM),
            pl.BlockSpec((H, 2 * H), lambda i: (jnp.int32(0), jnp.int32(0))),
            pl.BlockSpec((1, 2 * H), lambda i: (jnp.int32(0), jnp.int32(0))),
            pl.BlockSpec((2 * H, H), lambda i: (jnp.int32(0), jnp.int32(0))),
            pl.BlockSpec((1, H), lambda i: (jnp.int32(0), jnp.int32(0))),
            pl.BlockSpec((1, H), lambda i: (jnp.int32(0), jnp.int32(0))),
            pl.BlockSpec((1, H), lambda i: (jnp.int32(0), jnp.int32(0))),
        ],
        out_specs=pl.BlockSpec((blk, H), lambda i: (i, jnp.int32(0))),
        out_shape=jax.ShapeDtypeStruct((n, H), jnp.float32),
    )(h, a0, a1, scale, w1, b1, w2, b2, gamma, beta)


CH = 64
ROWS = E // CH
KCH = -(-ROWS // NW)
KCH = KCH + (-KCH % 4)
ROWS_PAD = NW * (KCH + 2)
TRASH = N


@functools.cache
def _build_agg():
    return functools.partial(
        pl.kernel,
        out_type=jax.ShapeDtypeStruct((NC, NPAD, H), jnp.float32),
        mesh=plsc.VectorSubcoreMesh(core_axis_name="c", subcore_axis_name="s",
                                    num_cores=NC, num_subcores=NS),
        scratch_types=[
            pltpu.VMEM((4, 2, CH), jnp.int32),
            pltpu.VMEM((CH, H), jnp.float32),
            pltpu.VMEM((CH, H), jnp.float32),
            pltpu.VMEM((CH, H), jnp.float32),
            pltpu.VMEM((CH, H), jnp.float32),
            pltpu.VMEM_SHARED((NPAD, H), jnp.float32),
            pltpu.SemaphoreType.DMA,
            pltpu.SemaphoreType.DMA,
            pltpu.SemaphoreType.DMA,
            pltpu.SemaphoreType.DMA,
            pltpu.SemaphoreType.DMA,
            pltpu.SemaphoreType.DMA,
            pltpu.SemaphoreType.DMA,
            pltpu.SemaphoreType.DMA,
        ],
    )(_agg_body)


def _agg_body(e_hbm, h_hbm, sd_hbm, zeros_hbm, out_hbm,
              idx4, e0, e1, g0, g1, agg_sh,
              si0, si1, se0, se1, sg0, sg1, sc0, sc1):
    c = lax.axis_index("c")
    s = lax.axis_index("s")
    wid = s * NC + c
    ebufs, gbufs = (e0, e1), (g0, g1)
    sis, ses, sgs, scs = (si0, si1), (se0, se1), (sg0, sg1), (sc0, sc1)
    emax = jnp.int32(E - CH)

    pltpu.sync_copy(zeros_hbm, agg_sh.at[pl.ds(s * (NPAD // NS), NPAD // NS)])
    plsc.subcore_barrier()

    def row_of(x):
        return wid + x * jnp.int32(NW)

    def issue_idx(x, j):
        pltpu.async_copy(sd_hbm.at[row_of(x)], idx4.at[jnp.int32(j)],
                         sis[j & 1])

    def issue_e(x, p):
        base = jnp.minimum(row_of(x) * jnp.int32(CH), emax)
        pltpu.async_copy(e_hbm.at[pl.ds(base, CH)], ebufs[p], ses[p])

    def issue_gather(x, j, p):
        pltpu.async_copy(h_hbm.at[idx4.at[jnp.int32(j), jnp.int32(0)]],
                         gbufs[p], sgs[p])

    def issue_scatter(j, p):
        pltpu.async_copy(gbufs[p], agg_sh.at[idx4.at[jnp.int32(j), jnp.int32(1)]],
                         scs[p], add=True)

    def wait_idx(j):
        pltpu.make_async_copy(sd_hbm.at[jnp.int32(0)], idx4.at[jnp.int32(j)],
                              sis[j & 1]).wait()

    def wait_e(p):
        pltpu.make_async_copy(e_hbm.at[pl.ds(0, CH)], ebufs[p], ses[p]).wait()

    def wait_g(p):
        pltpu.make_async_copy(e_hbm.at[pl.ds(0, CH)], gbufs[p], sgs[p]).wait()

    def wait_sc(p):
        pltpu.make_async_copy(gbufs[p], agg_sh.at[pl.ds(0, CH)], scs[p]).wait()

    def compute(p):
        gb, eb = gbufs[p], ebufs[p]

        def rowfn(i, cr):
            for jj in range(H // LANES):
                sl = pl.ds(jj * LANES, LANES)
                gb[i, sl] = jnp.maximum(gb[i, sl] + eb[i, sl], 0.0)
            return cr

        lax.fori_loop(jnp.int32(0), jnp.int32(CH), rowfn, jnp.int32(0))

    def process(x, j, first=False):
        p = j & 1
        q = 1 - p
        jn = (j + 1) & 3
        j2 = (j + 2) & 3
        wait_e(p)
        wait_g(p)
        compute(p)
        issue_scatter(j, p)
        issue_idx(x + 2, j2)
        issue_e(x + 2, p)
        wait_idx(jn)
        if not first:
            wait_sc(q)
        issue_gather(x + 1, jn, q)

    issue_idx(jnp.int32(0), 0)
    issue_e(jnp.int32(0), 0)
    wait_idx(0)
    issue_gather(jnp.int32(0), 0, 0)
    issue_idx(jnp.int32(1), 1)
    issue_e(jnp.int32(1), 1)

    process(jnp.int32(0), 0, first=True)
    process(jnp.int32(1), 1)
    process(jnp.int32(2), 2)
    process(jnp.int32(3), 3)

    def body(k, cr):
        x = k * jnp.int32(4)
        process(x, 0)
        process(x + 1, 1)
        process(x + 2, 2)
        process(x + 3, 3)
        return cr

    lax.fori_loop(jnp.int32(1), jnp.int32(KCH // 4), body, jnp.int32(0))

    wait_sc(1)
    wait_e(0)
    wait_e(1)
    wait_idx(1)
    wait_g(0)
    plsc.subcore_barrier()

    row0 = s * (NPAD // NS)
    pltpu.sync_copy(agg_sh.at[pl.ds(row0, NPAD // NS)],
                    out_hbm.at[c, pl.ds(row0, NPAD // NS)])


def kernel(x, ei, ea, W_proj, b_proj, eps, We, be, Wm1, bm1, Wm2, bm2,
           gamma, beta):
    src = ei[0].astype(jnp.int32).reshape(ROWS, CH)
    dst = ei[1].astype(jnp.int32).reshape(ROWS, CH)
    sd = jnp.stack([src, dst], axis=1)
    pad = jnp.full((ROWS_PAD - ROWS, 2, CH), TRASH, jnp.int32)
    pad = pad.at[:, 0, :].set(0)
    sd = jnp.concatenate([sd, pad], axis=0)
    zeros = jnp.zeros((NPAD // NS, H), jnp.float32)

    h = _proj(x, W_proj, b_proj.reshape(1, H), 1000)

    for l in range(L):
        e = _proj(ea, We[l], be[l].reshape(1, H), 4000)
        aggp = _build_agg()(e, h, sd, zeros)
        scale = (1.0 + eps[l]).reshape(1, 1).astype(jnp.float32)
        h = _node_update(h, aggp[0], aggp[1], scale, Wm1[l],
                         bm1[l].reshape(1, 2 * H), Wm2[l],
                         bm2[l].reshape(1, H), gamma[l].reshape(1, H),
                         beta[l].reshape(1, H), 1000)
    return h
